# Initial kernel scaffold; baseline (speedup 1.0000x reference)
import jax
import jax.numpy as jnp
from jax.experimental import pallas as pl


def kernel(edge_index, etypes, norm, basis0, w_comp0, basis1, w_comp1, basis2, w_comp2):
    raise NotImplementedError("write your pallas kernel here")


# speedup vs baseline: 2.5916x; 2.5916x over previous
import functools

import jax
import jax.numpy as jnp
from jax import lax
from jax.experimental import pallas as pl
from jax.experimental.pallas import tpu as pltpu
from jax.experimental.pallas import tpu_sc as plsc

N_NODES = 10000
N_RELS = 16
N_BASES = 8
H = 128
N_EDGES = 320000

NC = 2
NS = 16
NW = NC * NS
LANES = 16
CHUNK = 128
EPW = -(-N_EDGES // (NW * CHUNK)) * CHUNK
E_PAD = EPW * NW
NCHUNK = EPW // CHUNK
ZB = 80
NZB = N_NODES // ZB


def _wtable(X, BN):
    def body(wc_ref, basis_ref, out_ref):
        r = pl.program_id(1)
        acc = wc_ref[r, 0] * basis_ref[0]
        for b in range(1, N_BASES):
            acc = acc + wc_ref[r, b] * basis_ref[b]
        out_ref[0] = acc

    return pl.pallas_call(
        body,
        grid=(X // BN, N_RELS),
        in_specs=[
            pl.BlockSpec(memory_space=pltpu.SMEM),
            pl.BlockSpec((N_BASES, BN, H), lambda i, r: (0, i, 0)),
        ],
        out_specs=pl.BlockSpec((1, BN, H), lambda i, r: (r, i, 0)),
        out_shape=jax.ShapeDtypeStruct((N_RELS, X, H), jnp.float32),
    )


def _transform(BN):
    def body(p_ref, w_ref, out_ref):
        hblk = jnp.maximum(p_ref[0] + p_ref[1], 0.0)
        out_ref[0] = jnp.dot(hblk, w_ref[0], preferred_element_type=jnp.float32)

    return pl.pallas_call(
        body,
        grid=(N_NODES // BN, N_RELS),
        in_specs=[
            pl.BlockSpec((NC, BN, H), lambda i, r: (0, i, 0)),
            pl.BlockSpec((1, H, H), lambda i, r: (r, 0, 0)),
        ],
        out_specs=pl.BlockSpec((1, BN, H), lambda i, r: (r, i, 0)),
        out_shape=jax.ShapeDtypeStruct((N_RELS, N_NODES, H), jnp.float32),
    )


def _softmax_combine(BN):
    def body(p_ref, out_ref):
        h = p_ref[0] + p_ref[1]
        m = jnp.max(h, axis=1, keepdims=True)
        e = jnp.exp(h - m)
        out_ref[...] = e / jnp.sum(e, axis=1, keepdims=True)

    return pl.pallas_call(
        body,
        grid=(N_NODES // BN,),
        in_specs=[pl.BlockSpec((NC, BN, H), lambda i: (0, i, 0))],
        out_specs=pl.BlockSpec((BN, H), lambda i: (i, 0)),
        out_shape=jax.ShapeDtypeStruct((N_NODES, H), jnp.float32),
    )


_SC_AGG_CACHE = []


def _make_sc_agg():
    if _SC_AGG_CACHE:
        return _SC_AGG_CACHE[0]
    mesh = plsc.VectorSubcoreMesh(core_axis_name="c", subcore_axis_name="s",
                                  num_cores=NC, num_subcores=NS)

    @functools.partial(
        pl.kernel,
        out_type=jax.ShapeDtypeStruct((NC, N_NODES, H), jnp.float32),
        mesh=mesh,
        scratch_types=[
            pltpu.VMEM((CHUNK,), jnp.int32),
            pltpu.VMEM((CHUNK,), jnp.int32),
            pltpu.VMEM((CHUNK, LANES), jnp.float32),
            pltpu.VMEM((CHUNK, H), jnp.float32),
            pltpu.VMEM_SHARED((N_NODES, H), jnp.float32),
            pltpu.SemaphoreType.DMA,
        ],
    )
    def agg(table_hbm, idx_hbm, dst_hbm, norm_hbm, out_hbm,
            idx_v, dst_v, norm_v, rows_v, acc_sh, sem):
        cid = lax.axis_index("c")
        sid = lax.axis_index("s")
        wid = cid * NS + sid

        def zrow(i, _):
            for j in range(H // LANES):
                rows_v[i, pl.ds(j * LANES, LANES)] = jnp.zeros((LANES,), jnp.float32)
            return 0
        lax.fori_loop(0, ZB, zrow, 0)

        def zblk(k, _):
            blk = sid + NS * k

            @pl.when(blk < NZB)
            def _():
                pltpu.sync_copy(rows_v.at[pl.ds(0, ZB)],
                                acc_sh.at[pl.ds(blk * ZB, ZB)])
            return 0
        lax.fori_loop(0, -(-NZB // NS), zblk, 0)
        plsc.subcore_barrier()

        ebase = wid * EPW

        def chunk_body(c, _):
            base = ebase + c * CHUNK
            pltpu.sync_copy(idx_hbm.at[pl.ds(base, CHUNK)], idx_v)
            pltpu.sync_copy(dst_hbm.at[pl.ds(base, CHUNK)], dst_v)
            pltpu.sync_copy(norm_hbm.at[pl.ds(base, CHUNK)], norm_v)
            pltpu.async_copy(table_hbm.at[idx_v], rows_v, sem).wait()

            def scale(e, _):
                nv = norm_v[e]
                for j in range(H // LANES):
                    sl = rows_v[e, pl.ds(j * LANES, LANES)]
                    rows_v[e, pl.ds(j * LANES, LANES)] = sl * nv
                return 0
            lax.fori_loop(0, CHUNK, scale, 0)

            pltpu.sync_copy(rows_v, acc_sh.at[dst_v], add=True)
            return 0
        lax.fori_loop(0, NCHUNK, chunk_body, 0)

        plsc.subcore_barrier()

        def oblk(k, _):
            blk = sid + NS * k

            @pl.when(blk < NZB)
            def _():
                pltpu.sync_copy(acc_sh.at[pl.ds(blk * ZB, ZB)],
                                out_hbm.at[cid, pl.ds(blk * ZB, ZB)])
            return 0
        lax.fori_loop(0, -(-NZB // NS), oblk, 0)

    _SC_AGG_CACHE.append(agg)
    return agg


_wtable_big = _wtable(N_NODES, 1000)
_wtable_small = _wtable(H, H)
_transform_k = _transform(1000)
_softmax_k = _softmax_combine(1000)


def kernel(edge_index, etypes, norm, basis0, w_comp0, basis1, w_comp1,
           basis2, w_comp2):
    src = edge_index[0].astype(jnp.int32)
    dst = edge_index[1].astype(jnp.int32)
    et = etypes.astype(jnp.int32)
    idx = et * jnp.int32(N_NODES) + src

    pad = E_PAD - N_EDGES
    idx_p = jnp.pad(idx, (0, pad))
    dst_p = jnp.pad(dst, (0, pad))
    nrm_p = jnp.pad(norm.reshape(-1).astype(jnp.float32), (0, pad))
    norm16 = jnp.broadcast_to(nrm_p[:, None], (E_PAD, LANES))

    _sc_agg = _make_sc_agg()
    table0 = _wtable_big(w_comp0, basis0)
    w1full = _wtable_small(w_comp1, basis1)
    w2full = _wtable_small(w_comp2, basis2)

    p0 = _sc_agg(table0.reshape(N_RELS * N_NODES, H), idx_p, dst_p, norm16)
    t1 = _transform_k(p0, w1full)
    p1 = _sc_agg(t1.reshape(N_RELS * N_NODES, H), idx_p, dst_p, norm16)
    t2 = _transform_k(p1, w2full)
    p2 = _sc_agg(t2.reshape(N_RELS * N_NODES, H), idx_p, dst_p, norm16)
    return _softmax_k(p2)
